# baseline (speedup 1.0000x reference)
import functools

import jax
import jax.numpy as jnp
from jax import lax
from jax.experimental import pallas as pl
from jax.experimental.pallas import tpu as pltpu
from jax.experimental.pallas import tpu_sc as plsc

B = 4096
L = 200
EMBED = 128
SUB = 128
NUM_CLASSES = 10000

NC, NS = 2, 16
NW = NC * NS
RPW = B // NW
CHUNK = 104
NCH = 2
LPAD = CHUNK * NCH
VEC = 16


NBUF = 4
NCHK = RPW * NCH
NGRP = NCHK // NBUF


def _sc_pool(table, idx2):
    mesh = plsc.VectorSubcoreMesh(core_axis_name="c", subcore_axis_name="s")

    @functools.partial(
        pl.kernel,
        out_type=jax.ShapeDtypeStruct((B, EMBED), jnp.float32),
        mesh=mesh,
        scratch_types=[
            pltpu.VMEM((NCHK, CHUNK), jnp.int32),
            pltpu.VMEM((NBUF, CHUNK, EMBED), jnp.float32),
            pltpu.VMEM((RPW, EMBED), jnp.float32),
            [pltpu.SemaphoreType.DMA] * NBUF,
        ],
    )
    def k(table_hbm, idx_hbm, out_hbm, idx_v, gbuf, obuf, sems):
        wid = lax.axis_index("s") * NC + lax.axis_index("c")
        base = wid * RPW
        pltpu.sync_copy(idx_hbm.at[pl.ds(wid * NCHK, NCHK)], idx_v)

        def add_row(b, j, acc):
            return tuple(
                acc[kk] + gbuf[b, j, pl.ds(VEC * kk, VEC)]
                for kk in range(EMBED // VEC)
            )

        for b in range(NBUF):
            pltpu.async_copy(table_hbm.at[idx_v.at[b]], gbuf.at[b], sems[b])

        zeros = tuple(jnp.zeros((VEC,), jnp.float32) for _ in range(EMBED // VEC))
        scale = jnp.float32(1.0 / L)

        def group_body(g, acc):
            for b in range(NBUF):
                t = g * NBUF + b
                pltpu.make_async_copy(
                    table_hbm.at[idx_v.at[t]], gbuf.at[b], sems[b]
                ).wait()
                acc = lax.fori_loop(
                    0, CHUNK, functools.partial(add_row, b), acc
                )
                if b % NCH == NCH - 1:
                    r = g * (NBUF // NCH) + b // NCH
                    for kk in range(EMBED // VEC):
                        obuf[r, pl.ds(VEC * kk, VEC)] = acc[kk] * scale
                    acc = zeros
                nxt = t + NBUF

                @pl.when(nxt < NCHK)
                def _():
                    pltpu.async_copy(
                        table_hbm.at[idx_v.at[nxt]], gbuf.at[b], sems[b]
                    )
            return acc

        lax.fori_loop(0, NGRP, group_body, zeros)
        pltpu.sync_copy(obuf, out_hbm.at[pl.ds(base, RPW)])

    return k(table, idx2)


BM = 512
BN = 1024


def _mm_kernel(p_ref, s_ref, w_ref, b_ref, o_ref):
    p = p_ref[...].astype(jnp.bfloat16)
    s = s_ref[...].astype(jnp.bfloat16)
    w = w_ref[...].astype(jnp.bfloat16)
    dn = (((1,), (1,)), ((), ()))
    acc = lax.dot_general(p, w[:, :EMBED], dn,
                          preferred_element_type=jnp.float32)
    acc = acc + lax.dot_general(s, w[:, EMBED:], dn,
                                preferred_element_type=jnp.float32)
    o_ref[...] = acc + b_ref[...]


def _tc_classify(pooled, sub, W_cls, b_cls):
    grid = (B // BM, pl.cdiv(NUM_CLASSES, BN))
    return pl.pallas_call(
        _mm_kernel,
        grid=grid,
        in_specs=[
            pl.BlockSpec((BM, EMBED), lambda i, j: (i, 0)),
            pl.BlockSpec((BM, SUB), lambda i, j: (i, 0)),
            pl.BlockSpec((BN, EMBED + SUB), lambda i, j: (j, 0)),
            pl.BlockSpec((1, BN), lambda i, j: (0, j)),
        ],
        out_specs=pl.BlockSpec((BM, BN), lambda i, j: (i, j)),
        out_shape=jax.ShapeDtypeStruct((B, NUM_CLASSES), jnp.float32),
        compiler_params=pltpu.CompilerParams(
            dimension_semantics=("parallel", "parallel"),
        ),
    )(pooled, sub, W_cls, b_cls.reshape(1, NUM_CLASSES))


def kernel(word_input, sub_category_input, table, W_cls, b_cls):
    idx = word_input.astype(jnp.int32)
    idx = jnp.pad(idx, ((0, 0), (0, LPAD - L)))
    idx2 = idx.reshape(B * NCH, CHUNK)
    idx2 = jnp.zeros_like(idx2)
    pooled = _sc_pool(table, idx2)
    return _tc_classify(pooled, sub_category_input, W_cls, b_cls)


# speedup vs baseline: 66.0088x; 66.0088x over previous
import functools

import jax
import jax.numpy as jnp
from jax import lax
from jax.experimental import pallas as pl
from jax.experimental.pallas import tpu as pltpu
from jax.experimental.pallas import tpu_sc as plsc

B = 4096
L = 200
EMBED = 128
SUB = 128
NUM_CLASSES = 10000

NC, NS = 2, 16
NW = NC * NS
RPW = B // NW
CHUNK = 104
NCH = 2
LPAD = CHUNK * NCH
VEC = 16


NBUF = 4
NSPLIT = 4
BS = B // NSPLIT


def _sc_pool(table, idx2, bs):
    rpw = bs // NW
    nchk = rpw * NCH
    ngrp = nchk // NBUF
    mesh = plsc.VectorSubcoreMesh(core_axis_name="c", subcore_axis_name="s")

    @functools.partial(
        pl.kernel,
        out_type=jax.ShapeDtypeStruct((bs, EMBED), jnp.float32),
        mesh=mesh,
        scratch_types=[
            pltpu.VMEM((nchk, CHUNK), jnp.int32),
            pltpu.VMEM((NBUF, CHUNK, EMBED), jnp.float32),
            pltpu.VMEM((rpw, EMBED), jnp.float32),
            [pltpu.SemaphoreType.DMA] * NBUF,
        ],
    )
    def k(table_hbm, idx_hbm, out_hbm, idx_v, gbuf, obuf, sems):
        wid = lax.axis_index("s") * NC + lax.axis_index("c")
        base = wid * rpw
        pltpu.sync_copy(idx_hbm.at[pl.ds(wid * nchk, nchk)], idx_v)

        def add_row(b, j, acc):
            return tuple(
                acc[kk] + gbuf[b, j, pl.ds(VEC * kk, VEC)]
                for kk in range(EMBED // VEC)
            )

        for b in range(NBUF):
            pltpu.async_copy(table_hbm.at[idx_v.at[b]], gbuf.at[b], sems[b])

        zeros = tuple(jnp.zeros((VEC,), jnp.float32) for _ in range(EMBED // VEC))
        scale = jnp.float32(1.0 / L)

        def group_body(g, acc):
            for b in range(NBUF):
                t = g * NBUF + b
                pltpu.make_async_copy(
                    table_hbm.at[idx_v.at[t]], gbuf.at[b], sems[b]
                ).wait()
                acc = lax.fori_loop(
                    0, CHUNK, functools.partial(add_row, b), acc
                )
                if b % NCH == NCH - 1:
                    r = g * (NBUF // NCH) + b // NCH
                    for kk in range(EMBED // VEC):
                        obuf[r, pl.ds(VEC * kk, VEC)] = acc[kk] * scale
                    acc = zeros
                nxt = t + NBUF

                @pl.when(nxt < nchk)
                def _():
                    pltpu.async_copy(
                        table_hbm.at[idx_v.at[nxt]], gbuf.at[b], sems[b]
                    )
            return acc

        lax.fori_loop(0, ngrp, group_body, zeros)
        pltpu.sync_copy(obuf, out_hbm.at[pl.ds(base, rpw)])

    return k(table, idx2)


BM = 512
BN = 1024


def _mm_kernel(p_ref, s_ref, w_ref, b_ref, o_ref):
    p = p_ref[...].astype(jnp.bfloat16)
    s = s_ref[...].astype(jnp.bfloat16)
    w = w_ref[...].astype(jnp.bfloat16)
    dn = (((1,), (1,)), ((), ()))
    acc = lax.dot_general(p, w[:, :EMBED], dn,
                          preferred_element_type=jnp.float32)
    acc = acc + lax.dot_general(s, w[:, EMBED:], dn,
                                preferred_element_type=jnp.float32)
    o_ref[...] = acc + b_ref[...]


def _tc_classify(pooled, sub, W_cls, b_cls, bs):
    grid = (bs // BM, pl.cdiv(NUM_CLASSES, BN))
    return pl.pallas_call(
        _mm_kernel,
        grid=grid,
        in_specs=[
            pl.BlockSpec((BM, EMBED), lambda i, j: (i, 0)),
            pl.BlockSpec((BM, SUB), lambda i, j: (i, 0)),
            pl.BlockSpec((BN, EMBED + SUB), lambda i, j: (j, 0)),
            pl.BlockSpec((1, BN), lambda i, j: (0, j)),
        ],
        out_specs=pl.BlockSpec((BM, BN), lambda i, j: (i, j)),
        out_shape=jax.ShapeDtypeStruct((bs, NUM_CLASSES), jnp.float32),
        compiler_params=pltpu.CompilerParams(
            dimension_semantics=("parallel", "parallel"),
        ),
    )(pooled, sub, W_cls, b_cls.reshape(1, NUM_CLASSES))


def kernel(word_input, sub_category_input, table, W_cls, b_cls):
    idx = word_input.astype(jnp.int32)
    idx = jnp.pad(idx, ((0, 0), (0, LPAD - L)))
    idx2 = idx.reshape(B * NCH, CHUNK)
    outs = []
    for k in range(NSPLIT):
        i2 = idx2[k * BS * NCH:(k + 1) * BS * NCH]
        pooled = _sc_pool(table, i2, BS)
        sub = lax.slice_in_dim(sub_category_input, k * BS, (k + 1) * BS)
        outs.append(_tc_classify(pooled, sub, W_cls, b_cls, BS))
    return jnp.concatenate(outs, axis=0)
